# Initial kernel scaffold; baseline (speedup 1.0000x reference)
import jax
import jax.numpy as jnp
from jax.experimental import pallas as pl


def kernel(params, edge_index):
    raise NotImplementedError("write your pallas kernel here")


# speedup vs baseline: 197.1317x; 197.1317x over previous
import jax
import jax.numpy as jnp
from jax import lax
from jax.experimental import pallas as pl
from jax.experimental.pallas import tpu as pltpu
from jax.experimental.pallas import tpu_sc as plsc

N_AGENTS = 100000
N_EDGES = 3200000
N_STEPS = 10
BETA_SCALE = 1.0

NSUB = 16
PER = 6256
NPAD = NSUB * PER
NV = PER // 16
EPT = N_EDGES // NSUB
CH = 2000
NCH = EPT // CH


def _sir_sc(src_hbm, dst_hbm, u_hbm, pb_hbm, out_hbm,
            src_v, dst_v, val_v, one_v,
            inf_v, sus_v, deg_v, seg_v, ui_v, ur_v, zer_v,
            day_f, didx_v, pv, out_v,
            inf_sh, acc_sh, sem):
    wid = lax.axis_index("s")
    base = wid * PER
    ebase = wid * EPT
    z16 = jnp.zeros((16,), jnp.float32)
    io16 = lax.iota(jnp.int32, 16)

    pltpu.sync_copy(pb_hbm, pv)
    beta = pv[0]
    gamma = pv[1]
    frac = pv[2]

    def _zb(j, c):
        zer_v[pl.ds(j * 16, 16)] = z16
        return c
    lax.fori_loop(0, NV, _zb, 0)

    def _ob(j, c):
        one_v[pl.ds(j * 16, 16)] = z16 + 1.0
        return c
    lax.fori_loop(0, CH // 16, _ob, 0)

    pltpu.sync_copy(u_hbm.at[pl.ds(base, PER)], ui_v)

    def _ib(j, carry):
        a_i, a_s = carry
        sl = pl.ds(j * 16, 16)
        gid = base + j * 16 + io16
        m = gid < N_AGENTS
        u0 = ui_v[sl]
        i0 = jnp.where(frac < u0, 0.0, 1.0)
        s0 = jnp.where(m, 1.0 - i0, 0.0)
        inf_v[sl] = i0
        sus_v[sl] = s0
        return (a_i + i0, a_s + s0)
    a_i, a_s = lax.fori_loop(0, NV, _ib, (z16, z16))
    day_f[pl.ds(0, 16)] = a_i
    day_f[pl.ds(256, 16)] = a_s

    pltpu.sync_copy(inf_v, inf_sh.at[pl.ds(base, PER)])
    pltpu.sync_copy(zer_v, acc_sh.at[pl.ds(base, PER)])
    plsc.subcore_barrier()

    def _db(c, carry):
        off = ebase + c * CH
        pltpu.sync_copy(dst_hbm.at[pl.ds(off, CH)], dst_v)
        pltpu.sync_copy(one_v, acc_sh.at[dst_v], add=True)
        return carry
    lax.fori_loop(0, NCH, _db, 0)
    plsc.subcore_barrier()
    pltpu.sync_copy(acc_sh.at[pl.ds(base, PER)], deg_v)

    def _fb(j, c):
        sl = pl.ds(j * 16, 16)
        gid = base + j * 16 + io16
        m = gid < N_AGENTS
        deg_v[sl] = jnp.where(m, deg_v[sl], 1.0)
        return c
    lax.fori_loop(0, NV, _fb, 0)
    pltpu.sync_copy(zer_v, acc_sh.at[pl.ds(base, PER)])
    plsc.subcore_barrier()

    for t in range(N_STEPS):
        def _eb(c, carry):
            off = ebase + c * CH
            pltpu.sync_copy(src_hbm.at[pl.ds(off, CH)], src_v)
            pltpu.sync_copy(dst_hbm.at[pl.ds(off, CH)], dst_v)
            pltpu.async_copy(inf_sh.at[src_v], val_v, sem).wait()
            pltpu.sync_copy(val_v, acc_sh.at[dst_v], add=True)
            return carry
        lax.fori_loop(0, NCH, _eb, 0)
        plsc.subcore_barrier()

        pltpu.sync_copy(acc_sh.at[pl.ds(base, PER)], seg_v)
        pltpu.sync_copy(u_hbm.at[pl.ds((2 * t + 1) * NPAD + base, PER)], ui_v)
        pltpu.sync_copy(u_hbm.at[pl.ds((2 * t + 2) * NPAD + base, PER)], ur_v)

        def _ub(j, carry):
            a_i, a_s = carry
            sl = pl.ds(j * 16, 16)
            seg = seg_v[sl]
            dgg = deg_v[sl]
            s = sus_v[sl]
            i = inf_v[sl]
            x = beta * (s * seg) / dgg * BETA_SCALE
            ni = jnp.where(x < ui_v[sl], 0.0, 1.0)
            pr = jnp.minimum(jnp.maximum(gamma * i, 1e-10), 1.0)
            nr = jnp.where(pr < ur_v[sl], 0.0, 1.0)
            i2 = i + ni - nr
            s2 = s - ni
            inf_v[sl] = i2
            sus_v[sl] = s2
            return (a_i + i2, a_s + s2)
        a_i, a_s = lax.fori_loop(0, NV, _ub, (z16, z16))
        day_f[pl.ds(16 * (t + 1), 16)] = a_i
        day_f[pl.ds(256 + 16 * (t + 1), 16)] = a_s

        pltpu.sync_copy(zer_v, acc_sh.at[pl.ds(base, PER)])
        pltpu.sync_copy(inf_v, inf_sh.at[pl.ds(base, PER)])
        plsc.subcore_barrier()

    def _xb(j, c):
        didx_v[pl.ds(j * 16, 16)] = j * 16 + io16
        return c
    lax.fori_loop(0, 512 // 16, _xb, 0)
    pltpu.sync_copy(day_f, acc_sh.at[didx_v], add=True)
    plsc.subcore_barrier()

    @pl.when(wid == 0)
    def _():
        pltpu.sync_copy(acc_sh.at[pl.ds(0, 512)], day_f)
        row_s = z16
        row_i = z16
        row_r = z16
        n = jnp.float32(N_AGENTS)
        for d in range(N_STEPS + 1):
            vi = day_f[pl.ds(16 * d, 16)]
            vs = day_f[pl.ds(256 + 16 * d, 16)]
            ti = vi[0]
            ts = vs[0]
            for l in range(1, 16):
                ti = ti + vi[l]
                ts = ts + vs[l]
            m = io16 == d
            row_i = jnp.where(m, ti, row_i)
            row_s = jnp.where(m, ts, row_s)
            row_r = jnp.where(m, n - ti - ts, row_r)
        out_v[0] = row_s / n
        out_v[1] = row_i / n
        out_v[2] = row_r / n
        pltpu.sync_copy(out_v, out_hbm)


def _sir_call(src, dst, u_flat, pb):
    mesh = plsc.VectorSubcoreMesh(
        core_axis_name="c", subcore_axis_name="s", num_cores=1)
    f = pl.kernel(
        _sir_sc,
        out_type=jax.ShapeDtypeStruct((3, 16), jnp.float32),
        mesh=mesh,
        scratch_types=[
            pltpu.VMEM((CH,), jnp.int32),
            pltpu.VMEM((CH,), jnp.int32),
            pltpu.VMEM((CH,), jnp.float32),
            pltpu.VMEM((CH,), jnp.float32),
            pltpu.VMEM((PER,), jnp.float32),
            pltpu.VMEM((PER,), jnp.float32),
            pltpu.VMEM((PER,), jnp.float32),
            pltpu.VMEM((PER,), jnp.float32),
            pltpu.VMEM((PER,), jnp.float32),
            pltpu.VMEM((PER,), jnp.float32),
            pltpu.VMEM((PER,), jnp.float32),
            pltpu.VMEM((512,), jnp.float32),
            pltpu.VMEM((512,), jnp.int32),
            pltpu.VMEM((3, 16), jnp.float32),
            pltpu.VMEM((3, 16), jnp.float32),
            pltpu.VMEM_SHARED((NPAD,), jnp.float32),
            pltpu.VMEM_SHARED((NPAD,), jnp.float32),
            pltpu.SemaphoreType.DMA,
        ],
    )
    return f(src, dst, u_flat, pb)


def kernel(params, edge_index):
    src = edge_index[0]
    dst = edge_index[1]

    noise_key = jax.random.key(1234)
    us = []
    for i in range(2 * N_STEPS + 1):
        g = jax.random.gumbel(jax.random.fold_in(noise_key, i),
                              (N_AGENTS, 2), dtype=jnp.float32)
        c = g[:, 1] - g[:, 0]
        if i % 2 == 1:
            us.append(jax.nn.softplus(c))
        else:
            us.append(jax.nn.sigmoid(c))
    u = jnp.stack(us)
    pad = jnp.full((2 * N_STEPS + 1, NPAD - N_AGENTS), 3e38, jnp.float32)
    u_flat = jnp.concatenate([u, pad], axis=1).reshape(-1)

    pb = jnp.broadcast_to(params.astype(jnp.float32)[:, None], (3, 16))

    out = _sir_call(src, dst, u_flat, pb)
    return (out[0, :11], out[1, :11], out[2, :11])


# speedup vs baseline: 252.1255x; 1.2790x over previous
import jax
import jax.numpy as jnp
from jax import lax
from jax.experimental import pallas as pl
from jax.experimental.pallas import tpu as pltpu
from jax.experimental.pallas import tpu_sc as plsc

N_AGENTS = 100000
N_EDGES = 3200000
N_STEPS = 10
BETA_SCALE = 1.0

NSUB = 16
PER = 6256
NPAD = NSUB * PER
NV = PER // 16
EPT = N_EDGES // NSUB
CH = 4000
NCH = EPT // CH


def _sir_sc(src_hbm, dst_hbm, u_hbm, pb_hbm, out_hbm,
            src_v, dst_v, val_v, one_v,
            inf_v, sus_v, deg_v, seg_v, ui_v, ur_v, zer_v,
            day_f, didx_v, pv, out_v,
            inf_sh, acc_sh, sem):
    wid = lax.axis_index("s")
    base = wid * PER
    ebase = wid * EPT
    z16 = jnp.zeros((16,), jnp.float32)
    io16 = lax.iota(jnp.int32, 16)

    pltpu.sync_copy(pb_hbm, pv)
    beta = pv[0]
    gamma = pv[1]
    frac = pv[2]

    def _zb(j, c):
        zer_v[pl.ds(j * 16, 16)] = z16
        return c
    lax.fori_loop(0, NV, _zb, 0)

    def _ob(j, c):
        one_v[pl.ds(j * 16, 16)] = z16 + 1.0
        return c
    lax.fori_loop(0, CH // 16, _ob, 0)

    pltpu.sync_copy(u_hbm.at[pl.ds(base, PER)], ui_v)

    def _ib(j, carry):
        a_i, a_s = carry
        sl = pl.ds(j * 16, 16)
        gid = base + j * 16 + io16
        m = gid < N_AGENTS
        u0 = ui_v[sl]
        i0 = jnp.where(frac < u0, 0.0, 1.0)
        s0 = jnp.where(m, 1.0 - i0, 0.0)
        inf_v[sl] = i0
        sus_v[sl] = s0
        return (a_i + i0, a_s + s0)
    a_i, a_s = lax.fori_loop(0, NV, _ib, (z16, z16))
    day_f[pl.ds(0, 16)] = a_i
    day_f[pl.ds(256, 16)] = a_s

    pltpu.sync_copy(inf_v, inf_sh.at[pl.ds(base, PER)])
    pltpu.sync_copy(zer_v, acc_sh.at[pl.ds(base, PER)])
    plsc.subcore_barrier()

    def _db(c, carry):
        off = ebase + c * CH
        pltpu.sync_copy(dst_hbm.at[pl.ds(off, CH)], dst_v)
        pltpu.sync_copy(one_v, acc_sh.at[dst_v], add=True)
        return carry
    lax.fori_loop(0, NCH, _db, 0)
    plsc.subcore_barrier()
    pltpu.sync_copy(acc_sh.at[pl.ds(base, PER)], deg_v)

    def _fb(j, c):
        sl = pl.ds(j * 16, 16)
        gid = base + j * 16 + io16
        m = gid < N_AGENTS
        deg_v[sl] = jnp.where(m, deg_v[sl], 1.0)
        return c
    lax.fori_loop(0, NV, _fb, 0)
    pltpu.sync_copy(zer_v, acc_sh.at[pl.ds(base, PER)])
    plsc.subcore_barrier()

    for t in range(N_STEPS):
        def _eb(c, carry):
            off = ebase + c * CH
            pltpu.sync_copy(src_hbm.at[pl.ds(off, CH)], src_v)
            pltpu.sync_copy(dst_hbm.at[pl.ds(off, CH)], dst_v)
            pltpu.async_copy(inf_sh.at[src_v], val_v, sem).wait()
            pltpu.sync_copy(val_v, acc_sh.at[dst_v], add=True)
            return carry
        lax.fori_loop(0, NCH, _eb, 0)
        plsc.subcore_barrier()

        pltpu.sync_copy(acc_sh.at[pl.ds(base, PER)], seg_v)
        pltpu.sync_copy(u_hbm.at[pl.ds((2 * t + 1) * NPAD + base, PER)], ui_v)
        pltpu.sync_copy(u_hbm.at[pl.ds((2 * t + 2) * NPAD + base, PER)], ur_v)

        def _ub(j, carry):
            a_i, a_s = carry
            sl = pl.ds(j * 16, 16)
            seg = seg_v[sl]
            dgg = deg_v[sl]
            s = sus_v[sl]
            i = inf_v[sl]
            x = beta * (s * seg) / dgg * BETA_SCALE
            ni = jnp.where(x < ui_v[sl], 0.0, 1.0)
            pr = jnp.minimum(jnp.maximum(gamma * i, 1e-10), 1.0)
            nr = jnp.where(pr < ur_v[sl], 0.0, 1.0)
            i2 = i + ni - nr
            s2 = s - ni
            inf_v[sl] = i2
            sus_v[sl] = s2
            return (a_i + i2, a_s + s2)
        a_i, a_s = lax.fori_loop(0, NV, _ub, (z16, z16))
        day_f[pl.ds(16 * (t + 1), 16)] = a_i
        day_f[pl.ds(256 + 16 * (t + 1), 16)] = a_s

        pltpu.sync_copy(zer_v, acc_sh.at[pl.ds(base, PER)])
        pltpu.sync_copy(inf_v, inf_sh.at[pl.ds(base, PER)])
        plsc.subcore_barrier()

    def _xb(j, c):
        didx_v[pl.ds(j * 16, 16)] = j * 16 + io16
        return c
    lax.fori_loop(0, 512 // 16, _xb, 0)
    pltpu.sync_copy(day_f, acc_sh.at[didx_v], add=True)
    plsc.subcore_barrier()

    @pl.when(wid == 0)
    def _():
        pltpu.sync_copy(acc_sh.at[pl.ds(0, 512)], day_f)
        row_s = z16
        row_i = z16
        row_r = z16
        n = jnp.float32(N_AGENTS)
        for d in range(N_STEPS + 1):
            vi = day_f[pl.ds(16 * d, 16)]
            vs = day_f[pl.ds(256 + 16 * d, 16)]
            ti = vi[0]
            ts = vs[0]
            for l in range(1, 16):
                ti = ti + vi[l]
                ts = ts + vs[l]
            m = io16 == d
            row_i = jnp.where(m, ti, row_i)
            row_s = jnp.where(m, ts, row_s)
            row_r = jnp.where(m, n - ti - ts, row_r)
        out_v[0] = row_s / n
        out_v[1] = row_i / n
        out_v[2] = row_r / n
        pltpu.sync_copy(out_v, out_hbm)


def _sir_call(src, dst, u_flat, pb):
    mesh = plsc.VectorSubcoreMesh(
        core_axis_name="c", subcore_axis_name="s", num_cores=1)
    f = pl.kernel(
        _sir_sc,
        out_type=jax.ShapeDtypeStruct((3, 16), jnp.float32),
        mesh=mesh,
        scratch_types=[
            pltpu.VMEM((CH,), jnp.int32),
            pltpu.VMEM((CH,), jnp.int32),
            pltpu.VMEM((CH,), jnp.float32),
            pltpu.VMEM((CH,), jnp.float32),
            pltpu.VMEM((PER,), jnp.float32),
            pltpu.VMEM((PER,), jnp.float32),
            pltpu.VMEM((PER,), jnp.float32),
            pltpu.VMEM((PER,), jnp.float32),
            pltpu.VMEM((PER,), jnp.float32),
            pltpu.VMEM((PER,), jnp.float32),
            pltpu.VMEM((PER,), jnp.float32),
            pltpu.VMEM((512,), jnp.float32),
            pltpu.VMEM((512,), jnp.int32),
            pltpu.VMEM((3, 16), jnp.float32),
            pltpu.VMEM((3, 16), jnp.float32),
            pltpu.VMEM_SHARED((NPAD,), jnp.float32),
            pltpu.VMEM_SHARED((NPAD,), jnp.float32),
            pltpu.SemaphoreType.DMA,
        ],
    )
    return f(src, dst, u_flat, pb)


def kernel(params, edge_index):
    src = edge_index[0]
    dst = edge_index[1]

    noise_key = jax.random.key(1234)
    us = []
    for i in range(2 * N_STEPS + 1):
        g = jax.random.gumbel(jax.random.fold_in(noise_key, i),
                              (N_AGENTS, 2), dtype=jnp.float32)
        c = g[:, 1] - g[:, 0]
        if i % 2 == 1:
            us.append(jax.nn.softplus(c))
        else:
            us.append(jax.nn.sigmoid(c))
    u = jnp.stack(us)
    pad = jnp.full((2 * N_STEPS + 1, NPAD - N_AGENTS), 3e38, jnp.float32)
    u_flat = jnp.concatenate([u, pad], axis=1).reshape(-1)

    pb = jnp.broadcast_to(params.astype(jnp.float32)[:, None], (3, 16))

    out = _sir_call(src, dst, u_flat, pb)
    return (out[0, :11], out[1, :11], out[2, :11])


# speedup vs baseline: 256.2154x; 1.0162x over previous
import jax
import jax.numpy as jnp
from jax import lax
from jax.experimental import pallas as pl
from jax.experimental.pallas import tpu as pltpu
from jax.experimental.pallas import tpu_sc as plsc

N_AGENTS = 100000
N_EDGES = 3200000
N_STEPS = 10
BETA_SCALE = 1.0

NSUB = 16
PER = 6256
NPAD = NSUB * PER
NV = PER // 16
EPT = N_EDGES // NSUB
CH = 4000
NCH = EPT // CH


def _sir_sc(src_hbm, dst_hbm, u_hbm, pb_hbm, out_hbm,
            src_a, src_b, dst_a, dst_b, val_a, val_b, one_v, sla, slb, ssc,
            inf_v, sus_v, deg_v, seg_v, ui_v, ur_v, zer_v,
            day_f, didx_v, pv, out_v,
            inf_sh, acc_sh, sem):
    wid = lax.axis_index("s")
    base = wid * PER
    ebase = wid * EPT
    z16 = jnp.zeros((16,), jnp.float32)
    io16 = lax.iota(jnp.int32, 16)

    pltpu.sync_copy(pb_hbm, pv)
    beta = pv[0]
    gamma = pv[1]
    frac = pv[2]

    def _zb(j, c):
        zer_v[pl.ds(j * 16, 16)] = z16
        return c
    lax.fori_loop(0, NV, _zb, 0)

    def _ob(j, c):
        one_v[pl.ds(j * 16, 16)] = z16 + 1.0
        return c
    lax.fori_loop(0, CH // 16, _ob, 0)

    pltpu.sync_copy(u_hbm.at[pl.ds(base, PER)], ui_v)

    def _ib(j, carry):
        a_i, a_s = carry
        sl = pl.ds(j * 16, 16)
        gid = base + j * 16 + io16
        m = gid < N_AGENTS
        u0 = ui_v[sl]
        i0 = jnp.where(frac < u0, 0.0, 1.0)
        s0 = jnp.where(m, 1.0 - i0, 0.0)
        inf_v[sl] = i0
        sus_v[sl] = s0
        return (a_i + i0, a_s + s0)
    a_i, a_s = lax.fori_loop(0, NV, _ib, (z16, z16))
    day_f[pl.ds(0, 16)] = a_i
    day_f[pl.ds(256, 16)] = a_s

    pltpu.sync_copy(inf_v, inf_sh.at[pl.ds(base, PER)])
    pltpu.sync_copy(zer_v, acc_sh.at[pl.ds(base, PER)])
    plsc.subcore_barrier()

    def _db(c, carry):
        off = ebase + c * CH
        pltpu.sync_copy(dst_hbm.at[pl.ds(off, CH)], dst_a)
        pltpu.sync_copy(one_v, acc_sh.at[dst_a], add=True)
        return carry
    lax.fori_loop(0, NCH, _db, 0)
    plsc.subcore_barrier()
    pltpu.sync_copy(acc_sh.at[pl.ds(base, PER)], deg_v)

    def _fb(j, c):
        sl = pl.ds(j * 16, 16)
        gid = base + j * 16 + io16
        m = gid < N_AGENTS
        deg_v[sl] = jnp.where(m, deg_v[sl], 1.0)
        return c
    lax.fori_loop(0, NV, _fb, 0)
    pltpu.sync_copy(zer_v, acc_sh.at[pl.ds(base, PER)])
    plsc.subcore_barrier()

    for t in range(N_STEPS):
        pltpu.async_copy(src_hbm.at[pl.ds(ebase, CH)], src_a, sla)
        pltpu.async_copy(dst_hbm.at[pl.ds(ebase, CH)], dst_a, sla)

        def _pb(c2, carry):
            off0 = ebase + (2 * c2) * CH
            off1 = off0 + CH
            off2 = off1 + CH
            pltpu.make_async_copy(src_hbm.at[pl.ds(off0, CH)], src_a, sla).wait()
            pltpu.make_async_copy(dst_hbm.at[pl.ds(off0, CH)], dst_a, sla).wait()
            pltpu.async_copy(src_hbm.at[pl.ds(off1, CH)], src_b, slb)
            pltpu.async_copy(dst_hbm.at[pl.ds(off1, CH)], dst_b, slb)
            pltpu.async_copy(inf_sh.at[src_a], val_a, sem).wait()
            pltpu.async_copy(val_a, acc_sh.at[dst_a], ssc, add=True)
            pltpu.make_async_copy(src_hbm.at[pl.ds(off1, CH)], src_b, slb).wait()
            pltpu.make_async_copy(dst_hbm.at[pl.ds(off1, CH)], dst_b, slb).wait()
            pltpu.async_copy(inf_sh.at[src_b], val_b, sem).wait()
            pltpu.make_async_copy(val_a, acc_sh.at[dst_a], ssc).wait()

            @pl.when(c2 + 1 < NCH // 2)
            def _():
                pltpu.async_copy(src_hbm.at[pl.ds(off2, CH)], src_a, sla)
                pltpu.async_copy(dst_hbm.at[pl.ds(off2, CH)], dst_a, sla)
            pltpu.async_copy(val_b, acc_sh.at[dst_b], ssc, add=True)
            pltpu.make_async_copy(val_b, acc_sh.at[dst_b], ssc).wait()
            return carry
        lax.fori_loop(0, NCH // 2, _pb, 0)
        plsc.subcore_barrier()

        pltpu.sync_copy(acc_sh.at[pl.ds(base, PER)], seg_v)
        pltpu.sync_copy(u_hbm.at[pl.ds((2 * t + 1) * NPAD + base, PER)], ui_v)
        pltpu.sync_copy(u_hbm.at[pl.ds((2 * t + 2) * NPAD + base, PER)], ur_v)

        def _ub(j, carry):
            a_i, a_s = carry
            sl = pl.ds(j * 16, 16)
            seg = seg_v[sl]
            dgg = deg_v[sl]
            s = sus_v[sl]
            i = inf_v[sl]
            x = beta * (s * seg) / dgg * BETA_SCALE
            ni = jnp.where(x < ui_v[sl], 0.0, 1.0)
            pr = jnp.minimum(jnp.maximum(gamma * i, 1e-10), 1.0)
            nr = jnp.where(pr < ur_v[sl], 0.0, 1.0)
            i2 = i + ni - nr
            s2 = s - ni
            inf_v[sl] = i2
            sus_v[sl] = s2
            return (a_i + i2, a_s + s2)
        a_i, a_s = lax.fori_loop(0, NV, _ub, (z16, z16))
        day_f[pl.ds(16 * (t + 1), 16)] = a_i
        day_f[pl.ds(256 + 16 * (t + 1), 16)] = a_s

        pltpu.sync_copy(zer_v, acc_sh.at[pl.ds(base, PER)])
        pltpu.sync_copy(inf_v, inf_sh.at[pl.ds(base, PER)])
        plsc.subcore_barrier()

    def _xb(j, c):
        didx_v[pl.ds(j * 16, 16)] = j * 16 + io16
        return c
    lax.fori_loop(0, 512 // 16, _xb, 0)
    pltpu.sync_copy(day_f, acc_sh.at[didx_v], add=True)
    plsc.subcore_barrier()

    @pl.when(wid == 0)
    def _():
        pltpu.sync_copy(acc_sh.at[pl.ds(0, 512)], day_f)
        row_s = z16
        row_i = z16
        row_r = z16
        n = jnp.float32(N_AGENTS)
        for d in range(N_STEPS + 1):
            vi = day_f[pl.ds(16 * d, 16)]
            vs = day_f[pl.ds(256 + 16 * d, 16)]
            ti = vi[0]
            ts = vs[0]
            for l in range(1, 16):
                ti = ti + vi[l]
                ts = ts + vs[l]
            m = io16 == d
            row_i = jnp.where(m, ti, row_i)
            row_s = jnp.where(m, ts, row_s)
            row_r = jnp.where(m, n - ti - ts, row_r)
        out_v[0] = row_s / n
        out_v[1] = row_i / n
        out_v[2] = row_r / n
        pltpu.sync_copy(out_v, out_hbm)


def _sir_call(src, dst, u_flat, pb):
    mesh = plsc.VectorSubcoreMesh(
        core_axis_name="c", subcore_axis_name="s", num_cores=1)
    f = pl.kernel(
        _sir_sc,
        out_type=jax.ShapeDtypeStruct((3, 16), jnp.float32),
        mesh=mesh,
        scratch_types=[
            pltpu.VMEM((CH,), jnp.int32),
            pltpu.VMEM((CH,), jnp.int32),
            pltpu.VMEM((CH,), jnp.int32),
            pltpu.VMEM((CH,), jnp.int32),
            pltpu.VMEM((CH,), jnp.float32),
            pltpu.VMEM((CH,), jnp.float32),
            pltpu.VMEM((CH,), jnp.float32),
            pltpu.SemaphoreType.DMA,
            pltpu.SemaphoreType.DMA,
            pltpu.SemaphoreType.DMA,
            pltpu.VMEM((PER,), jnp.float32),
            pltpu.VMEM((PER,), jnp.float32),
            pltpu.VMEM((PER,), jnp.float32),
            pltpu.VMEM((PER,), jnp.float32),
            pltpu.VMEM((PER,), jnp.float32),
            pltpu.VMEM((PER,), jnp.float32),
            pltpu.VMEM((PER,), jnp.float32),
            pltpu.VMEM((512,), jnp.float32),
            pltpu.VMEM((512,), jnp.int32),
            pltpu.VMEM((3, 16), jnp.float32),
            pltpu.VMEM((3, 16), jnp.float32),
            pltpu.VMEM_SHARED((NPAD,), jnp.float32),
            pltpu.VMEM_SHARED((NPAD,), jnp.float32),
            pltpu.SemaphoreType.DMA,
        ],
    )
    return f(src, dst, u_flat, pb)


def kernel(params, edge_index):
    src = edge_index[0]
    dst = edge_index[1]

    noise_key = jax.random.key(1234)
    us = []
    for i in range(2 * N_STEPS + 1):
        g = jax.random.gumbel(jax.random.fold_in(noise_key, i),
                              (N_AGENTS, 2), dtype=jnp.float32)
        c = g[:, 1] - g[:, 0]
        if i % 2 == 1:
            us.append(jax.nn.softplus(c))
        else:
            us.append(jax.nn.sigmoid(c))
    u = jnp.stack(us)
    pad = jnp.full((2 * N_STEPS + 1, NPAD - N_AGENTS), 3e38, jnp.float32)
    u_flat = jnp.concatenate([u, pad], axis=1).reshape(-1)

    pb = jnp.broadcast_to(params.astype(jnp.float32)[:, None], (3, 16))

    out = _sir_call(src, dst, u_flat, pb)
    return (out[0, :11], out[1, :11], out[2, :11])
